# baseline (speedup 1.0000x reference)
import jax
import jax.numpy as jnp
from jax.experimental import pallas as pl
from jax.experimental.pallas import tpu as pltpu

N = 4096
HID = 2048
BATCH = 64

GB = 1
BB = BATCH // GB
K = 128
HC = 128


def _nade_kernel(xb_ref, xt_ref, wtb_ref, v_ref, b_ref, ch_ref, out_ref,
                 a_ref):
    nb = pl.program_id(1)

    @pl.when(nb == 0)
    def _init():
        a_ref[...] = jnp.broadcast_to(ch_ref[...], (BB, HID))
        out_ref[...] = jnp.zeros_like(out_ref)

    xb = xb_ref[...]
    xt = xt_ref[0]
    wtb = wtb_ref[...]
    a0 = a_ref[...]

    rows = jax.lax.broadcasted_iota(jnp.int32, (K, 1, K), 0)
    cols = jax.lax.broadcasted_iota(jnp.int32, (K, 1, K), 2)
    lhalf = jnp.where(rows > cols, jnp.float32(0.5), jnp.float32(0.0))
    m2 = (lhalf * xb[None]).astype(jnp.bfloat16)
    m2r = m2.reshape(K * BB, K)

    tvacc = jnp.zeros((K * BB, 128), jnp.float32)
    vsacc = jnp.zeros((K, 128), jnp.float32)
    for ci in range(HID // HC):
        sl = slice(ci * HC, (ci + 1) * HC)
        v_c = v_ref[:, sl]
        zh = jnp.dot(m2r, wtb[:, sl],
                     preferred_element_type=jnp.float32)
        zh = zh + jnp.tile(a0[:, sl], (K, 1))
        t = jnp.tanh(zh)
        vb = jnp.broadcast_to(v_c[:, None, :],
                              (K, BB, HC)).reshape(K * BB, HC)
        tv = t * vb
        for j in range(HC // 128):
            tvacc = tvacc + tv[:, j * 128:(j + 1) * 128]
            vsacc = vsacc + v_c[:, j * 128:(j + 1) * 128]

    tvs = jnp.sum(tvacc.reshape(K, BB, 128), axis=-1)
    vss = jnp.sum(vsacc, axis=-1, keepdims=True)
    logits = b_ref[0] + 0.5 * vss + 0.5 * tvs

    lp = xt * jax.nn.log_sigmoid(logits) \
        + (1.0 - xt) * jax.nn.log_sigmoid(-logits)
    out_ref[...] += jnp.sum(lp, axis=0, keepdims=True)[None]

    xth = (xt * 0.5).astype(jnp.bfloat16)
    a_ref[...] = a0 + jax.lax.dot_general(
        xth, wtb, (((0,), (0,)), ((), ())),
        preferred_element_type=jnp.float32)


def kernel(x, W, c, V, b):
    xt = x.T.reshape(N, GB, BB).transpose(1, 0, 2)
    wtb = W.T.astype(jnp.bfloat16)
    ch = (0.5 * c).reshape(1, HID)
    b3 = b.reshape(N // K, K, 1)
    out = pl.pallas_call(
        _nade_kernel,
        grid=(GB, N // K),
        in_specs=[
            pl.BlockSpec((BB, K), lambda g, nb: (g, nb)),
            pl.BlockSpec((1, K, BB), lambda g, nb: (g, nb, 0)),
            pl.BlockSpec((K, HID), lambda g, nb: (nb, 0)),
            pl.BlockSpec((K, HID), lambda g, nb: (nb, 0)),
            pl.BlockSpec((1, K, 1), lambda g, nb: (nb, 0, 0)),
            pl.BlockSpec((1, HID), lambda g, nb: (0, 0)),
        ],
        out_specs=pl.BlockSpec((1, 1, BB), lambda g, nb: (g, 0, 0)),
        out_shape=jax.ShapeDtypeStruct((GB, 1, BB), jnp.float32),
        scratch_shapes=[pltpu.VMEM((BB, HID), jnp.float32)],
        compiler_params=pltpu.CompilerParams(
            dimension_semantics=("parallel", "arbitrary"),
            vmem_limit_bytes=56 * 1024 * 1024,
        ),
        name="nade_fwd",
    )(x, xt, wtb, V, b3, ch)
    return out.reshape(BATCH)


# speedup vs baseline: 1.0131x; 1.0131x over previous
import jax
import jax.numpy as jnp
from jax.experimental import pallas as pl
from jax.experimental.pallas import tpu as pltpu

N = 4096
HID = 2048
BATCH = 64

GB = 1
BB = BATCH // GB
K = 128
HC = 512


def _nade_kernel(xb_ref, xt_ref, wtb_ref, v_ref, b_ref, ch_ref, out_ref,
                 a_ref):
    nb = pl.program_id(1)

    @pl.when(nb == 0)
    def _init():
        a_ref[...] = jnp.broadcast_to(ch_ref[...], (BB, HID))
        out_ref[...] = jnp.zeros_like(out_ref)

    xb = xb_ref[...]
    xt = xt_ref[0]
    wtb = wtb_ref[...]
    a0 = a_ref[...]

    rows = jax.lax.broadcasted_iota(jnp.int32, (K, 1, K), 0)
    cols = jax.lax.broadcasted_iota(jnp.int32, (K, 1, K), 2)
    lhalf = jnp.where(rows > cols, jnp.float32(0.5), jnp.float32(0.0))
    m2 = (lhalf * xb[None]).astype(jnp.bfloat16)
    m2r = m2.reshape(K * BB, K)

    tvacc = jnp.zeros((K * BB, 128), jnp.float32)
    vsacc = jnp.zeros((K, 128), jnp.float32)
    for ci in range(HID // HC):
        sl = slice(ci * HC, (ci + 1) * HC)
        v_c = v_ref[:, sl]
        zh = jnp.dot(m2r, wtb[:, sl],
                     preferred_element_type=jnp.float32)
        zh = zh + jnp.tile(a0[:, sl], (K, 1))
        t = jnp.tanh(zh)
        vb = jnp.broadcast_to(v_c[:, None, :],
                              (K, BB, HC)).reshape(K * BB, HC)
        tv = t * vb
        for j in range(HC // 128):
            tvacc = tvacc + tv[:, j * 128:(j + 1) * 128]
            vsacc = vsacc + v_c[:, j * 128:(j + 1) * 128]

    tvs = jnp.sum(tvacc.reshape(K, BB, 128), axis=-1)
    vss = jnp.sum(vsacc, axis=-1, keepdims=True)
    logits = b_ref[0] + 0.5 * vss + 0.5 * tvs

    lp = xt * jax.nn.log_sigmoid(logits) \
        + (1.0 - xt) * jax.nn.log_sigmoid(-logits)
    out_ref[...] += jnp.sum(lp, axis=0, keepdims=True)[None]

    xth = (xt * 0.5).astype(jnp.bfloat16)
    a_ref[...] = a0 + jax.lax.dot_general(
        xth, wtb, (((0,), (0,)), ((), ())),
        preferred_element_type=jnp.float32)


def kernel(x, W, c, V, b):
    xt = x.T.reshape(N, GB, BB).transpose(1, 0, 2)
    wtb = W.T.astype(jnp.bfloat16)
    ch = (0.5 * c).reshape(1, HID)
    b3 = b.reshape(N // K, K, 1)
    out = pl.pallas_call(
        _nade_kernel,
        grid=(GB, N // K),
        in_specs=[
            pl.BlockSpec((BB, K), lambda g, nb: (g, nb)),
            pl.BlockSpec((1, K, BB), lambda g, nb: (g, nb, 0)),
            pl.BlockSpec((K, HID), lambda g, nb: (nb, 0)),
            pl.BlockSpec((K, HID), lambda g, nb: (nb, 0)),
            pl.BlockSpec((1, K, 1), lambda g, nb: (nb, 0, 0)),
            pl.BlockSpec((1, HID), lambda g, nb: (0, 0)),
        ],
        out_specs=pl.BlockSpec((1, 1, BB), lambda g, nb: (g, 0, 0)),
        out_shape=jax.ShapeDtypeStruct((GB, 1, BB), jnp.float32),
        scratch_shapes=[pltpu.VMEM((BB, HID), jnp.float32)],
        compiler_params=pltpu.CompilerParams(
            dimension_semantics=("parallel", "arbitrary"),
            vmem_limit_bytes=56 * 1024 * 1024,
        ),
        name="nade_fwd",
    )(x, xt, wtb, V, b3, ch)
    return out.reshape(BATCH)


# speedup vs baseline: 1.0972x; 1.0830x over previous
import jax
import jax.numpy as jnp
from jax.experimental import pallas as pl
from jax.experimental.pallas import tpu as pltpu

N = 4096
HID = 2048
BATCH = 64

GB = 1
BB = BATCH // GB
K = 128
HC = 256


def _nade_kernel(xb_ref, xt_ref, wtb_ref, v_ref, b_ref, ch_ref, out_ref,
                 a_ref):
    nb = pl.program_id(1)

    @pl.when(nb == 0)
    def _init():
        a_ref[...] = jnp.broadcast_to(ch_ref[...], (BB, HID))
        out_ref[...] = jnp.zeros_like(out_ref)

    xb = xb_ref[...]
    xt = xt_ref[0]
    wtb = wtb_ref[...]
    a0 = a_ref[...]

    rows = jax.lax.broadcasted_iota(jnp.int32, (K, 1, K), 0)
    cols = jax.lax.broadcasted_iota(jnp.int32, (K, 1, K), 2)
    lhalf = jnp.where(rows > cols, jnp.float32(0.5), jnp.float32(0.0))
    m2 = (lhalf * xb[None]).astype(jnp.bfloat16)
    m2r = m2.reshape(K * BB, K)

    tvacc = jnp.zeros((K * BB, 128), jnp.bfloat16)
    vsacc = jnp.zeros((K, 128), jnp.float32)
    for ci in range(HID // HC):
        sl = slice(ci * HC, (ci + 1) * HC)
        v_c = v_ref[:, sl]
        zh = jnp.dot(m2r, wtb[:, sl],
                     preferred_element_type=jnp.float32)
        zh = zh + jnp.tile(a0[:, sl], (K, 1))
        t = jnp.tanh(zh).astype(jnp.bfloat16)
        vb = jnp.broadcast_to(v_c[:, None, :],
                              (K, BB, HC)).reshape(K * BB, HC)
        tv = t * vb.astype(jnp.bfloat16)
        for j in range(HC // 128):
            tvacc = tvacc + tv[:, j * 128:(j + 1) * 128]
            vsacc = vsacc + v_c[:, j * 128:(j + 1) * 128]

    tvs = jnp.sum(tvacc.astype(jnp.float32).reshape(K, BB, 128),
                  axis=-1)
    vss = jnp.sum(vsacc, axis=-1, keepdims=True)
    logits = b_ref[0] + 0.5 * vss + 0.5 * tvs

    lp = xt * jax.nn.log_sigmoid(logits) \
        + (1.0 - xt) * jax.nn.log_sigmoid(-logits)
    out_ref[...] += jnp.sum(lp, axis=0, keepdims=True)[None]

    xth = (xt * 0.5).astype(jnp.bfloat16)
    a_ref[...] = a0 + jax.lax.dot_general(
        xth, wtb, (((0,), (0,)), ((), ())),
        preferred_element_type=jnp.float32)


def kernel(x, W, c, V, b):
    xt = x.T.reshape(N, GB, BB).transpose(1, 0, 2)
    wtb = W.T.astype(jnp.bfloat16)
    ch = (0.5 * c).reshape(1, HID)
    b3 = b.reshape(N // K, K, 1)
    out = pl.pallas_call(
        _nade_kernel,
        grid=(GB, N // K),
        in_specs=[
            pl.BlockSpec((BB, K), lambda g, nb: (g, nb)),
            pl.BlockSpec((1, K, BB), lambda g, nb: (g, nb, 0)),
            pl.BlockSpec((K, HID), lambda g, nb: (nb, 0)),
            pl.BlockSpec((K, HID), lambda g, nb: (nb, 0)),
            pl.BlockSpec((1, K, 1), lambda g, nb: (nb, 0, 0)),
            pl.BlockSpec((1, HID), lambda g, nb: (0, 0)),
        ],
        out_specs=pl.BlockSpec((1, 1, BB), lambda g, nb: (g, 0, 0)),
        out_shape=jax.ShapeDtypeStruct((GB, 1, BB), jnp.float32),
        scratch_shapes=[pltpu.VMEM((BB, HID), jnp.float32)],
        compiler_params=pltpu.CompilerParams(
            dimension_semantics=("parallel", "arbitrary"),
            vmem_limit_bytes=56 * 1024 * 1024,
        ),
        name="nade_fwd",
    )(x, xt, wtb, V, b3, ch)
    return out.reshape(BATCH)
